# Initial kernel scaffold; baseline (speedup 1.0000x reference)
import jax
import jax.numpy as jnp
from jax.experimental import pallas as pl


def kernel(img):
    raise NotImplementedError("write your pallas kernel here")


# speedup vs baseline: 2713.1972x; 2713.1972x over previous
import jax
import jax.numpy as jnp
from jax.experimental import pallas as pl
from jax.experimental.pallas import tpu as pltpu

EPS_ = 1e-7
H_ = 224
W_ = 224
CW_ = 222
C19 = 0.1111111111111111
DELTA_ = EPS_ / 9.0

OFFS = [(di, dj) for di in range(3) for dj in range(3)]
PAIRS = [(0, 0), (0, 1), (0, 2), (1, 1), (1, 2), (2, 2)]


def _bf(x):
    return x.astype(jnp.bfloat16).astype(jnp.float32)


def _red9(ts):
    a0 = (ts[0] + ts[8]) + ts[4]
    b0 = ts[2] + ts[6]
    c0 = ts[1] + ts[5]
    d0 = ts[3] + ts[7]
    return (a0 + b0) + (c0 + d0)


def _tree9(ts):
    return (((ts[0] + ts[1]) + (ts[2] + ts[3]))
            + ((ts[4] + ts[5]) + (ts[6] + ts[7]))) + ts[8]


def _loss_kernel(img_ref, out_ref):
    c19 = jnp.float32(C19)
    delta = jnp.float32(DELTA_)

    P = [img_ref[c] for c in range(3)]
    Pb = [_bf(p) for p in P]
    xs = [[p[di:di + CW_, dj:dj + CW_] for (di, dj) in OFFS] for p in P]
    xb = [[p[di:di + CW_, dj:dj + CW_] for (di, dj) in OFFS] for p in Pb]

    mu = [_red9(xs[c]) * c19 for c in range(3)]

    A = {}
    for (i, k) in PAIRS:
        g = _tree9([xb[i][j] * xb[k][j] for j in range(9)])
        a = g * c19 - mu[i] * mu[k]
        if i == k:
            a = a + delta
        A[(i, k)] = a
        A[(k, i)] = a

    rows = [[A[(r, 0)], A[(r, 1)], A[(r, 2)]] for r in range(3)]
    ab0 = [jnp.abs(rows[r][0]) for r in range(3)]
    p1 = ab0[1] > ab0[0]
    p2 = ab0[2] > jnp.maximum(ab0[0], ab0[1])
    def pick(r0, r1, r2):
        return jnp.where(p2, r2, jnp.where(p1, r1, r0))
    nr0 = [pick(rows[0][c], rows[1][c], rows[2][c]) for c in range(3)]
    nr1 = [jnp.where(p1 & ~p2, rows[0][c], rows[1][c]) for c in range(3)]
    nr2 = [jnp.where(p2, rows[0][c], rows[2][c]) for c in range(3)]
    i0 = jnp.zeros_like(rows[0][0], dtype=jnp.int32)
    i1 = jnp.ones_like(i0)
    i2 = i1 + i1
    q0 = jnp.where(p2, i2, jnp.where(p1, i1, i0))
    q1 = jnp.where(p1 & ~p2, i0, i1)
    q2 = jnp.where(p2, i0, i2)

    u00, u01, u02 = nr0
    l10 = nr1[0] / u00
    l20 = nr2[0] / u00
    a11 = nr1[1] - l10 * u01
    a12 = nr1[2] - l10 * u02
    a21 = nr2[1] - l20 * u01
    a22 = nr2[2] - l20 * u02
    s2 = jnp.abs(a21) > jnp.abs(a11)
    u11 = jnp.where(s2, a21, a11)
    u12 = jnp.where(s2, a22, a12)
    b21 = jnp.where(s2, a11, a21)
    b22 = jnp.where(s2, a12, a22)
    l10_, l20_ = jnp.where(s2, l20, l10), jnp.where(s2, l10, l20)
    nq1, nq2 = jnp.where(s2, q2, q1), jnp.where(s2, q1, q2)
    l21 = b21 / u11
    u22 = b22 - l21 * u12
    l10, l20 = l10_, l20_
    q1, q2 = nq1, nq2

    one = jnp.float32(1.0)
    Linv = [[one, None, None],
            [-l10, one, None],
            [(l10 * l21) - l20, -l21, one]]
    r0 = one / u00
    r1 = one / u11
    r2 = one / u22
    s01 = u01 / u00
    s02 = u02 / u00
    s12 = u12 / u11
    Uinv = [[r0, -(s01 * r1), ((s01 * s12) - s02) * r2],
            [None, r1, -(s12 * r2)],
            [None, None, r2]]

    def ycol(k, j):
        jj = jnp.full_like(i0, j)
        zero = jnp.zeros_like(one * i0.astype(jnp.float32))
        v0 = Linv[k][0] if k >= 0 else None
        def ent(kk, rr):
            v = Linv[kk][rr]
            if v is None:
                return zero
            return v
        return jnp.where(q0 == jj, ent(k, 0),
                         jnp.where(q1 == jj, ent(k, 1), ent(k, 2)))

    inv = [[None] * 3 for _ in range(3)]
    for i in range(3):
        for j in range(3):
            terms = []
            for k in range(i, 3):
                terms.append(Uinv[i][k] * ycol(k, j))
            acc = terms[0]
            for tt in terms[1:]:
                acc = acc + tt
            inv[i][j] = _bf(acc)

    D = [[_bf(xs[k][j] - mu[k]) for k in range(3)] for j in range(9)]
    X = [[None] * 3 for _ in range(9)]
    for j in range(9):
        for i in range(3):
            acc = D[j][0] * inv[0][i]
            acc = acc + D[j][1] * inv[1][i]
            acc = acc + D[j][2] * inv[2][i]
            X[j][i] = _bf(acc)

    total = None
    for c in range(3):
        s = _red9(xs[c])
        t = [_bf(_tree9([D[j][k] * xb[c][j] for j in range(9)]))
             for k in range(3)]
        for j in range(9):
            u = X[j][0] * t[0]
            u = u + X[j][1] * t[1]
            u = u + X[j][2] * t[2]
            yw = xs[c][j] - (s + u) * c19
            contrib = yw * xs[c][j]
            total = contrib if total is None else total + contrib

    out_ref[0, 0] = jnp.sum(total)


def kernel(img):
    planes = img[0]
    out = pl.pallas_call(
        _loss_kernel,
        out_shape=jax.ShapeDtypeStruct((1, 1), jnp.float32),
        out_specs=pl.BlockSpec(memory_space=pltpu.SMEM),
    )(planes)
    return out[0, 0]


# speedup vs baseline: 2721.5390x; 1.0031x over previous
import jax
import jax.numpy as jnp
from jax.experimental import pallas as pl
from jax.experimental.pallas import tpu as pltpu

EPS_ = 1e-7
H_ = 224
W_ = 224
CW_ = 222
C19 = 0.1111111111111111
DELTA_ = EPS_ / 9.0

OFFS = [(di, dj) for di in range(3) for dj in range(3)]
PAIRS = [(0, 0), (0, 1), (0, 2), (1, 1), (1, 2), (2, 2)]


def _bf(x):
    return x.astype(jnp.bfloat16).astype(jnp.float32)


def _red9(ts):
    a0 = (ts[0] + ts[8]) + ts[4]
    b0 = ts[2] + ts[6]
    c0 = ts[1] + ts[5]
    d0 = ts[3] + ts[7]
    return (a0 + b0) + (c0 + d0)


def _tree9(ts):
    return (((ts[0] + ts[1]) + (ts[2] + ts[3]))
            + ((ts[4] + ts[5]) + (ts[6] + ts[7]))) + ts[8]


def _loss_kernel(img_ref, out_ref):
    c19 = jnp.float32(C19)
    delta = jnp.float32(DELTA_)

    P = [img_ref[c] for c in range(3)]
    Pb = [_bf(p) for p in P]
    xs = [[p[di:di + CW_, dj:dj + CW_] for (di, dj) in OFFS] for p in P]
    xb = [[p[di:di + CW_, dj:dj + CW_] for (di, dj) in OFFS] for p in Pb]

    mu = [_red9(xs[c]) * c19 for c in range(3)]

    A = {}
    for (i, k) in PAIRS:
        g = _tree9([xb[i][j] * xb[k][j] for j in range(9)])
        a = g * c19 - mu[i] * mu[k]
        if i == k:
            a = a + delta
        A[(i, k)] = a
        A[(k, i)] = a

    rows = [[A[(r, 0)], A[(r, 1)], A[(r, 2)]] for r in range(3)]
    ab0 = [jnp.abs(rows[r][0]) for r in range(3)]
    p1 = ab0[1] > ab0[0]
    p2 = ab0[2] > jnp.maximum(ab0[0], ab0[1])
    def pick(r0, r1, r2):
        return jnp.where(p2, r2, jnp.where(p1, r1, r0))
    nr0 = [pick(rows[0][c], rows[1][c], rows[2][c]) for c in range(3)]
    nr1 = [jnp.where(p1 & ~p2, rows[0][c], rows[1][c]) for c in range(3)]
    nr2 = [jnp.where(p2, rows[0][c], rows[2][c]) for c in range(3)]
    i0 = jnp.zeros_like(rows[0][0], dtype=jnp.int32)
    i1 = jnp.ones_like(i0)
    i2 = i1 + i1
    q0 = jnp.where(p2, i2, jnp.where(p1, i1, i0))
    q1 = jnp.where(p1 & ~p2, i0, i1)
    q2 = jnp.where(p2, i0, i2)

    u00, u01, u02 = nr0
    l10 = nr1[0] / u00
    l20 = nr2[0] / u00
    a11 = nr1[1] - l10 * u01
    a12 = nr1[2] - l10 * u02
    a21 = nr2[1] - l20 * u01
    a22 = nr2[2] - l20 * u02
    s2 = jnp.abs(a21) > jnp.abs(a11)
    u11 = jnp.where(s2, a21, a11)
    u12 = jnp.where(s2, a22, a12)
    b21 = jnp.where(s2, a11, a21)
    b22 = jnp.where(s2, a12, a22)
    l10_, l20_ = jnp.where(s2, l20, l10), jnp.where(s2, l10, l20)
    nq1, nq2 = jnp.where(s2, q2, q1), jnp.where(s2, q1, q2)
    l21 = b21 / u11
    u22 = b22 - l21 * u12
    l10, l20 = l10_, l20_
    q1, q2 = nq1, nq2

    one = jnp.float32(1.0)
    Linv = [[one, None, None],
            [-l10, one, None],
            [(l10 * l21) - l20, -l21, one]]
    r0 = one / u00
    r1 = one / u11
    r2 = one / u22
    s01 = u01 / u00
    s02 = u02 / u00
    s12 = u12 / u11
    Uinv = [[r0, -(s01 * r1), ((s01 * s12) - s02) * r2],
            [None, r1, -(s12 * r2)],
            [None, None, r2]]

    def ycol(k, j):
        jj = jnp.full_like(i0, j)
        zero = jnp.zeros_like(u00)
        def ent(kk, rr):
            v = Linv[kk][rr]
            return zero if v is None else v
        return jnp.where(q0 == jj, ent(k, 0),
                         jnp.where(q1 == jj, ent(k, 1), ent(k, 2)))

    inv = [[None] * 3 for _ in range(3)]
    for i in range(3):
        for j in range(3):
            terms = []
            for k in range(i, 3):
                terms.append(Uinv[i][k] * ycol(k, j))
            acc = terms[0]
            for tt in terms[1:]:
                acc = acc + tt
            inv[i][j] = _bf(acc)

    D = [[_bf(xs[k][j] - mu[k]) for k in range(3)] for j in range(9)]
    X = [[None] * 3 for _ in range(9)]
    for j in range(9):
        for i in range(3):
            acc = D[j][0] * inv[0][i]
            acc = acc + D[j][1] * inv[1][i]
            acc = acc + D[j][2] * inv[2][i]
            X[j][i] = _bf(acc)

    total = None
    for c in range(3):
        s = _red9(xs[c])
        t = [_bf(_tree9([D[j][k] * xb[c][j] for j in range(9)]))
             for k in range(3)]
        for j in range(9):
            u = X[j][0] * t[0]
            u = u + X[j][1] * t[1]
            u = u + X[j][2] * t[2]
            yw = xs[c][j] - (s + u) * c19
            contrib = yw * xs[c][j]
            total = contrib if total is None else total + contrib

    out_ref[0, 0] = jnp.sum(total)


def kernel(img):
    planes = img[0]
    out = pl.pallas_call(
        _loss_kernel,
        out_shape=jax.ShapeDtypeStruct((1, 1), jnp.float32),
        out_specs=pl.BlockSpec(memory_space=pltpu.SMEM),
    )(planes)
    return out[0, 0]
